# baseline (speedup 1.0000x reference)
import functools

import jax
import jax.numpy as jnp
from jax import lax
from jax.experimental import pallas as pl
from jax.experimental.pallas import tpu as pltpu
from jax.experimental.pallas import tpu_sc as plsc

N = 2048
K = 32
H = 128
R = N * K
FF = 4 * H
SCALE = 30.0

NODE_BLK = 256
EDGE_BLK = NODE_BLK * K
NBLK = N // NODE_BLK
PARTS = 8
PBLK = NBLK // PARTS

CHUNK = 128
NWORKERS = 32
NCHUNK = R // CHUNK
CPW = NCHUNK // PARTS // NWORKERS

_SQRT_HALF = 0.7071067811865476


def _gelu(x):
    return 0.5 * x * (1.0 + lax.erf(x * _SQRT_HALF))


def _gelu16(x):
    x = x.astype(jnp.bfloat16)
    return (jnp.bfloat16(0.5) * x
            * (jnp.bfloat16(1.0)
               + lax.erf(x * jnp.bfloat16(_SQRT_HALF))))


def _dot16(a, b):
    return jnp.dot(a.astype(jnp.bfloat16), b.astype(jnp.bfloat16),
                   preferred_element_type=jnp.float32)


def _ln(x, g, b):
    mu = jnp.mean(x, axis=-1, keepdims=True)
    xc = x - mu
    var = jnp.mean(xc * xc, axis=-1, keepdims=True)
    return xc / jnp.sqrt(var + 1e-5) * g + b


def _prep_body(hv_ref, w_ref, q_ref):
    q_ref[...] = jnp.dot(hv_ref[...], w_ref[...])


def _tc_prep(hv, w1c):
    return pl.pallas_call(
        _prep_body,
        out_shape=jax.ShapeDtypeStruct((N, H), jnp.float32),
    )(hv, w1c)


def _sc_gather(table, idx2d, part):
    NBUF = 3
    chunk0 = part * (NCHUNK // PARTS)

    @functools.partial(
        pl.kernel,
        mesh=plsc.VectorSubcoreMesh(core_axis_name="c", subcore_axis_name="s"),
        out_type=jax.ShapeDtypeStruct((R // PARTS, H), jnp.float32),
        scratch_types=[
            pltpu.VMEM((CPW, CHUNK), jnp.int32),
            pltpu.VMEM((NBUF, CHUNK, H), jnp.float32),
            pltpu.SemaphoreType.DMA,
            pltpu.SemaphoreType.DMA((NBUF,)),
            pltpu.SemaphoreType.DMA((NBUF,)),
        ],
    )
    def k(table_hbm, idx_hbm, out_hbm, idx_v, rows_v, sem_i, sem_g, sem_w):
        wid = lax.axis_index("s") * 2 + lax.axis_index("c")
        base = wid * CPW
        pltpu.async_copy(idx_hbm.at[pl.ds(chunk0 + base, CPW)], idx_v,
                         sem_i).wait()

        def start_gather(t):
            return pltpu.async_copy(table_hbm.at[idx_v.at[t]],
                                    rows_v.at[t % NBUF], sem_g.at[t % NBUF])

        gh = {t: start_gather(t) for t in range(min(NBUF, CPW))}
        wh = {}
        for t in range(CPW):
            b = t % NBUF
            gh[t].wait()
            wh[t] = pltpu.async_copy(
                rows_v.at[b], out_hbm.at[pl.ds((base + t) * CHUNK, CHUNK)],
                sem_w.at[b])
            if t + NBUF < CPW:
                wh[t].wait()
                gh[t + NBUF] = start_gather(t + NBUF)
        for t in range(max(0, CPW - NBUF), CPW):
            wh[t].wait()

    return k(table, idx2d)


def _main_body(hv_ref, he_ref, g1_ref, ma_ref, mv_ref,
               w1a_ref, b1_ref, w1b_ref, w2_ref, b2_ref, w3_ref, b3_ref,
               wdin_ref, bdin_ref, wdout_ref, bdout_ref,
               ln1g_ref, ln1b_ref, ln2g_ref, ln2b_ref,
               w11a_ref, b11_ref, w11c_ref,
               hv2_ref, q2_ref, pre2_ref):
    hv = hv_ref[...]
    pre1 = _dot16(hv, w1a_ref[...]) + b1_ref[...]
    he16 = he_ref[...].reshape(EDGE_BLK, H)
    x = _dot16(he16, w1b_ref[...]) + g1_ref[...]
    x = (x.reshape(NODE_BLK, K, H) + pre1[:, None, :]).reshape(EDGE_BLK, H)
    x = _gelu16(x)
    x = _gelu16(_dot16(x, w2_ref[...]) + b2_ref[...])
    m = _dot16(x, w3_ref[...]) + b3_ref[...]
    m3 = m.reshape(NODE_BLK, K, H) * ma_ref[...].reshape(NODE_BLK, K)[:, :, None]
    dh = jnp.sum(m3, axis=1) * (1.0 / SCALE)
    hv2 = _ln(hv + dh, ln1g_ref[...], ln1b_ref[...])
    ffn = _dot16(_gelu16(_dot16(hv2, wdin_ref[...]) + bdin_ref[...]),
                 wdout_ref[...]) + bdout_ref[...]
    hv2 = _ln(hv2 + ffn, ln2g_ref[...], ln2b_ref[...])
    hv2 = hv2 * mv_ref[...]
    hv2_ref[...] = hv2
    q2_ref[...] = jnp.dot(hv2, w11c_ref[...])
    pre2_ref[...] = _dot16(hv2, w11a_ref[...]) + b11_ref[...]


def _tc_main(hv, he4, g1h, ma, mv, weights, part):
    off = part * PBLK
    node_h = pl.BlockSpec((NODE_BLK, H), lambda i, off=off: (i + off, 0))
    he_spec = pl.BlockSpec((1, NODE_BLK, K, H),
                           lambda i, off=off: (0, i + off, 0, 0))
    gh_spec = pl.BlockSpec((EDGE_BLK, H), lambda i: (i, 0))
    out_node = pl.BlockSpec((NODE_BLK, H), lambda i: (i, 0))

    def full(a):
        return pl.BlockSpec(a.shape, lambda i: (0,) * a.ndim)

    in_specs = [
        node_h, he_spec, gh_spec,
        pl.BlockSpec((1, NODE_BLK, K), lambda i, off=off: (0, i + off, 0)),
        pl.BlockSpec((NODE_BLK, 1), lambda i, off=off: (i + off, 0)),
    ] + [full(w) for w in weights]
    out_specs = [out_node, out_node, out_node]
    out_shape = [jax.ShapeDtypeStruct((N // PARTS, H), jnp.float32)] * 3
    return pl.pallas_call(
        _main_body,
        grid=(PBLK,),
        in_specs=in_specs,
        out_specs=out_specs,
        out_shape=out_shape,
        compiler_params=pltpu.CompilerParams(
            dimension_semantics=("arbitrary",)),
    )(hv, he4, g1h, ma, mv, *weights)


def _edge_body(he_ref, g2_ref, pre2_ref, w11b_ref, w12_ref, b12_ref,
               w13_ref, b13_ref, ln3g_ref, ln3b_ref, he2_ref):
    he16 = he_ref[...].reshape(EDGE_BLK, H)
    x = _dot16(he16, w11b_ref[...]) + g2_ref[...]
    x = (x.reshape(NODE_BLK, K, H) + pre2_ref[...][:, None, :]).reshape(EDGE_BLK, H)
    x = _gelu16(x)
    x = _gelu16(_dot16(x, w12_ref[...]) + b12_ref[...])
    m = _dot16(x, w13_ref[...]) + b13_ref[...]
    he2 = _ln(he16.astype(jnp.float32) + m, ln3g_ref[...], ln3b_ref[...])
    he2_ref[...] = he2.reshape(1, NODE_BLK, K, H)


def _edge_body_aliased(_alias_ref, *rest):
    _edge_body(*rest)


def _tc_edge(he4, g2p, pre2p, weights, part, he2_prev):
    off = part * PBLK
    he_spec = pl.BlockSpec((1, NODE_BLK, K, H),
                           lambda i, off=off: (0, i + off, 0, 0))
    gh_spec = pl.BlockSpec((EDGE_BLK, H), lambda i: (i, 0))
    node_h = pl.BlockSpec((NODE_BLK, H), lambda i: (i, 0))
    out_spec = pl.BlockSpec((1, NODE_BLK, K, H),
                            lambda i, off=off: (0, i + off, 0, 0))

    def full(a):
        return pl.BlockSpec(a.shape, lambda i: (0,) * a.ndim)

    in_specs = [he_spec, gh_spec, node_h] + [full(w) for w in weights]
    body = _edge_body
    args = (he4, g2p, pre2p) + tuple(weights)
    aliases = {}
    if he2_prev is not None:
        in_specs = [pl.BlockSpec(memory_space=pltpu.MemorySpace.HBM)] + in_specs
        body = _edge_body_aliased
        args = (he2_prev,) + args
        aliases = {0: 0}
    return pl.pallas_call(
        body,
        grid=(PBLK,),
        in_specs=in_specs,
        out_specs=out_spec,
        out_shape=jax.ShapeDtypeStruct((1, N, K, H), jnp.float32),
        input_output_aliases=aliases,
        compiler_params=pltpu.CompilerParams(
            dimension_semantics=("arbitrary",)),
    )(*args)


def kernel(h_V, h_E, mask_V, mask_attend, W1, b1, W2, b2, W3, b3,
           Wd_in, bd_in, Wd_out, bd_out, W11, b11, W12, b12, W13, b13,
           ln1_g, ln1_b, ln2_g, ln2_b, ln3_g, ln3_b, E_idx):
    hv = h_V.reshape(N, H)
    ma = mask_attend
    mv = mask_V.reshape(N, 1)
    idx2d = E_idx.reshape(NCHUNK, CHUNK).astype(jnp.int32)

    w1a, w1b, w1c = W1[:H], W1[H:2 * H], W1[2 * H:]
    w11a, w11b, w11c = W11[:H], W11[H:2 * H], W11[2 * H:]
    row = lambda v: v.reshape(1, -1)

    main_w = (w1a, row(b1), w1b, W2, row(b2), W3, row(b3),
              Wd_in, row(bd_in), Wd_out, row(bd_out),
              row(ln1_g), row(ln1_b), row(ln2_g), row(ln2_b),
              w11a, row(b11), w11c)
    edge_w = (w11b, W12, row(b12), W13, row(b13), row(ln3_g), row(ln3_b))

    q1 = _tc_prep(hv, w1c)
    g1 = [_sc_gather(q1, idx2d, p) for p in range(PARTS)]
    mains = [_tc_main(hv, h_E, g1[p], ma, mv, main_w, p)
             for p in range(PARTS)]
    q2 = jnp.concatenate([m[1] for m in mains], axis=0)
    g2 = [_sc_gather(q2, idx2d, p) for p in range(PARTS)]
    he2 = None
    for p in range(PARTS):
        he2 = _tc_edge(h_E, g2[p], mains[p][2], edge_w, p, he2)
    hv2 = jnp.concatenate([m[0] for m in mains], axis=0)
    return (hv2.reshape(1, N, H), he2)


# speedup vs baseline: 1.1204x; 1.1204x over previous
import functools

import jax
import jax.numpy as jnp
from jax import lax
from jax.experimental import pallas as pl
from jax.experimental.pallas import tpu as pltpu
from jax.experimental.pallas import tpu_sc as plsc

N = 2048
K = 32
H = 128
R = N * K
FF = 4 * H
SCALE = 30.0

NODE_BLK = 128
EDGE_BLK = NODE_BLK * K
NBLK = N // NODE_BLK
PARTS = 4
PBLK = NBLK // PARTS

CHUNK = 128
NWORKERS = 32
NCHUNK = R // CHUNK
CPW = NCHUNK // PARTS // NWORKERS

_SQRT_HALF = 0.7071067811865476


def _gelu(x):
    return 0.5 * x * (1.0 + lax.erf(x * _SQRT_HALF))


def _gelu16(x):
    x = x.astype(jnp.bfloat16)
    return (jnp.bfloat16(0.5) * x
            * (jnp.bfloat16(1.0)
               + lax.erf(x * jnp.bfloat16(_SQRT_HALF))))


def _dot16(a, b):
    return jnp.dot(a.astype(jnp.bfloat16), b.astype(jnp.bfloat16),
                   preferred_element_type=jnp.float32)


def _ln(x, g, b):
    mu = jnp.mean(x, axis=-1, keepdims=True)
    xc = x - mu
    var = jnp.mean(xc * xc, axis=-1, keepdims=True)
    return xc / jnp.sqrt(var + 1e-5) * g + b


def _prep_body(hv_ref, w_ref, q_ref):
    q_ref[...] = jnp.dot(hv_ref[...], w_ref[...])


def _tc_prep(hv, w1c):
    return pl.pallas_call(
        _prep_body,
        out_shape=jax.ShapeDtypeStruct((N, H), jnp.float32),
    )(hv, w1c)


def _sc_gather(table, idx2d, part):
    NBUF = 3
    chunk0 = part * (NCHUNK // PARTS)

    @functools.partial(
        pl.kernel,
        mesh=plsc.VectorSubcoreMesh(core_axis_name="c", subcore_axis_name="s"),
        out_type=jax.ShapeDtypeStruct((R // PARTS, H), jnp.float32),
        scratch_types=[
            pltpu.VMEM((CPW, CHUNK), jnp.int32),
            pltpu.VMEM((NBUF, CHUNK, H), jnp.float32),
            pltpu.SemaphoreType.DMA,
            pltpu.SemaphoreType.DMA((NBUF,)),
            pltpu.SemaphoreType.DMA((NBUF,)),
        ],
    )
    def k(table_hbm, idx_hbm, out_hbm, idx_v, rows_v, sem_i, sem_g, sem_w):
        wid = lax.axis_index("s") * 2 + lax.axis_index("c")
        base = wid * CPW
        pltpu.async_copy(idx_hbm.at[pl.ds(chunk0 + base, CPW)], idx_v,
                         sem_i).wait()

        def start_gather(t):
            return pltpu.async_copy(table_hbm.at[idx_v.at[t]],
                                    rows_v.at[t % NBUF], sem_g.at[t % NBUF])

        gh = {t: start_gather(t) for t in range(min(NBUF, CPW))}
        wh = {}
        for t in range(CPW):
            b = t % NBUF
            gh[t].wait()
            wh[t] = pltpu.async_copy(
                rows_v.at[b], out_hbm.at[pl.ds((base + t) * CHUNK, CHUNK)],
                sem_w.at[b])
            if t + NBUF < CPW:
                wh[t].wait()
                gh[t + NBUF] = start_gather(t + NBUF)
        for t in range(max(0, CPW - NBUF), CPW):
            wh[t].wait()

    return k(table, idx2d)


def _main_body(hv_ref, he_ref, g1_ref, ma_ref, mv_ref,
               w1a_ref, b1_ref, w1b_ref, w2_ref, b2_ref, w3_ref, b3_ref,
               wdin_ref, bdin_ref, wdout_ref, bdout_ref,
               ln1g_ref, ln1b_ref, ln2g_ref, ln2b_ref,
               w11a_ref, b11_ref, w11c_ref,
               hv2_ref, q2_ref, pre2_ref):
    hv = hv_ref[...]
    pre1 = _dot16(hv, w1a_ref[...]) + b1_ref[...]
    he16 = he_ref[...].reshape(EDGE_BLK, H)
    x = _dot16(he16, w1b_ref[...]) + g1_ref[...]
    x = (x.reshape(NODE_BLK, K, H) + pre1[:, None, :]).reshape(EDGE_BLK, H)
    x = _gelu16(x)
    x = _gelu16(_dot16(x, w2_ref[...]) + b2_ref[...])
    m = _dot16(x, w3_ref[...]) + b3_ref[...]
    m3 = m.reshape(NODE_BLK, K, H) * ma_ref[...].reshape(NODE_BLK, K)[:, :, None]
    dh = jnp.sum(m3, axis=1) * (1.0 / SCALE)
    hv2 = _ln(hv + dh, ln1g_ref[...], ln1b_ref[...])
    ffn = _dot16(_gelu16(_dot16(hv2, wdin_ref[...]) + bdin_ref[...]),
                 wdout_ref[...]) + bdout_ref[...]
    hv2 = _ln(hv2 + ffn, ln2g_ref[...], ln2b_ref[...])
    hv2 = hv2 * mv_ref[...]
    hv2_ref[...] = hv2
    q2_ref[...] = jnp.dot(hv2, w11c_ref[...])
    pre2_ref[...] = _dot16(hv2, w11a_ref[...]) + b11_ref[...]


def _tc_main(hv, he4, g1h, ma, mv, weights, part):
    off = part * PBLK
    node_h = pl.BlockSpec((NODE_BLK, H), lambda i, off=off: (i + off, 0))
    he_spec = pl.BlockSpec((1, NODE_BLK, K, H),
                           lambda i, off=off: (0, i + off, 0, 0))
    gh_spec = pl.BlockSpec((EDGE_BLK, H), lambda i: (i, 0))
    out_node = pl.BlockSpec((NODE_BLK, H), lambda i: (i, 0))

    def full(a):
        return pl.BlockSpec(a.shape, lambda i: (0,) * a.ndim)

    in_specs = [
        node_h, he_spec, gh_spec,
        pl.BlockSpec((1, NODE_BLK, K), lambda i, off=off: (0, i + off, 0)),
        pl.BlockSpec((NODE_BLK, 1), lambda i, off=off: (i + off, 0)),
    ] + [full(w) for w in weights]
    out_specs = [out_node, out_node, out_node]
    out_shape = [jax.ShapeDtypeStruct((N // PARTS, H), jnp.float32)] * 3
    return pl.pallas_call(
        _main_body,
        grid=(PBLK,),
        in_specs=in_specs,
        out_specs=out_specs,
        out_shape=out_shape,
        compiler_params=pltpu.CompilerParams(
            dimension_semantics=("arbitrary",)),
    )(hv, he4, g1h, ma, mv, *weights)


def _edge_body(he_ref, g2_ref, pre2_ref, w11b_ref, w12_ref, b12_ref,
               w13_ref, b13_ref, ln3g_ref, ln3b_ref, he2_ref):
    he16 = he_ref[...].reshape(EDGE_BLK, H)
    x = _dot16(he16, w11b_ref[...]) + g2_ref[...]
    x = (x.reshape(NODE_BLK, K, H) + pre2_ref[...][:, None, :]).reshape(EDGE_BLK, H)
    x = _gelu16(x)
    x = _gelu16(_dot16(x, w12_ref[...]) + b12_ref[...])
    m = _dot16(x, w13_ref[...]) + b13_ref[...]
    he2 = _ln(he16.astype(jnp.float32) + m, ln3g_ref[...], ln3b_ref[...])
    he2_ref[...] = he2.reshape(1, NODE_BLK, K, H)


def _edge_body_aliased(_alias_ref, *rest):
    _edge_body(*rest)


def _tc_edge(he4, g2p, pre2p, weights, part, he2_prev):
    off = part * PBLK
    he_spec = pl.BlockSpec((1, NODE_BLK, K, H),
                           lambda i, off=off: (0, i + off, 0, 0))
    gh_spec = pl.BlockSpec((EDGE_BLK, H), lambda i: (i, 0))
    node_h = pl.BlockSpec((NODE_BLK, H), lambda i: (i, 0))
    out_spec = pl.BlockSpec((1, NODE_BLK, K, H),
                            lambda i, off=off: (0, i + off, 0, 0))

    def full(a):
        return pl.BlockSpec(a.shape, lambda i: (0,) * a.ndim)

    in_specs = [he_spec, gh_spec, node_h] + [full(w) for w in weights]
    body = _edge_body
    args = (he4, g2p, pre2p) + tuple(weights)
    aliases = {}
    if he2_prev is not None:
        in_specs = [pl.BlockSpec(memory_space=pltpu.MemorySpace.HBM)] + in_specs
        body = _edge_body_aliased
        args = (he2_prev,) + args
        aliases = {0: 0}
    return pl.pallas_call(
        body,
        grid=(PBLK,),
        in_specs=in_specs,
        out_specs=out_spec,
        out_shape=jax.ShapeDtypeStruct((1, N, K, H), jnp.float32),
        input_output_aliases=aliases,
        compiler_params=pltpu.CompilerParams(
            dimension_semantics=("arbitrary",)),
    )(*args)


def kernel(h_V, h_E, mask_V, mask_attend, W1, b1, W2, b2, W3, b3,
           Wd_in, bd_in, Wd_out, bd_out, W11, b11, W12, b12, W13, b13,
           ln1_g, ln1_b, ln2_g, ln2_b, ln3_g, ln3_b, E_idx):
    hv = h_V.reshape(N, H)
    ma = mask_attend
    mv = mask_V.reshape(N, 1)
    idx2d = E_idx.reshape(NCHUNK, CHUNK).astype(jnp.int32)

    w1a, w1b, w1c = W1[:H], W1[H:2 * H], W1[2 * H:]
    w11a, w11b, w11c = W11[:H], W11[H:2 * H], W11[2 * H:]
    row = lambda v: v.reshape(1, -1)

    main_w = (w1a, row(b1), w1b, W2, row(b2), W3, row(b3),
              Wd_in, row(bd_in), Wd_out, row(bd_out),
              row(ln1_g), row(ln1_b), row(ln2_g), row(ln2_b),
              w11a, row(b11), w11c)
    edge_w = (w11b, W12, row(b12), W13, row(b13), row(ln3_g), row(ln3_b))

    q1 = _tc_prep(hv, w1c)
    g1 = [_sc_gather(q1, idx2d, p) for p in range(PARTS)]
    mains = [_tc_main(hv, h_E, g1[p], ma, mv, main_w, p)
             for p in range(PARTS)]
    q2 = jnp.concatenate([m[1] for m in mains], axis=0)
    g2 = [_sc_gather(q2, idx2d, p) for p in range(PARTS)]
    he2 = None
    for p in range(PARTS):
        he2 = _tc_edge(h_E, g2[p], mains[p][2], edge_w, p, he2)
    hv2 = jnp.concatenate([m[0] for m in mains], axis=0)
    return (hv2.reshape(1, N, H), he2)


# speedup vs baseline: 1.2062x; 1.0766x over previous
import functools

import jax
import jax.numpy as jnp
from jax import lax
from jax.experimental import pallas as pl
from jax.experimental.pallas import tpu as pltpu
from jax.experimental.pallas import tpu_sc as plsc

N = 2048
K = 32
H = 128
R = N * K
FF = 4 * H
SCALE = 30.0

NODE_BLK = 256
EDGE_BLK = NODE_BLK * K
NBLK = N // NODE_BLK
PARTS = 4
PBLK = NBLK // PARTS

CHUNK = 128
NWORKERS = 32
NCHUNK = R // CHUNK
CPW = NCHUNK // PARTS // NWORKERS

_SQRT_HALF = 0.7071067811865476


def _gelu(x):
    return 0.5 * x * (1.0 + lax.erf(x * _SQRT_HALF))


def _gelu16(x):
    x = x.astype(jnp.bfloat16)
    return (jnp.bfloat16(0.5) * x
            * (jnp.bfloat16(1.0)
               + lax.erf(x * jnp.bfloat16(_SQRT_HALF))))


def _dot16(a, b):
    return jnp.dot(a.astype(jnp.bfloat16), b.astype(jnp.bfloat16),
                   preferred_element_type=jnp.float32)


def _ln(x, g, b):
    mu = jnp.mean(x, axis=-1, keepdims=True)
    s2 = jnp.mean(x * x, axis=-1, keepdims=True)
    r = lax.rsqrt(s2 - mu * mu + 1e-5)
    return (x - mu) * r * g + b


def _prep_body(hv_ref, w_ref, q_ref):
    q_ref[...] = jnp.dot(hv_ref[...], w_ref[...])


def _tc_prep(hv, w1c):
    return pl.pallas_call(
        _prep_body,
        out_shape=jax.ShapeDtypeStruct((N, H), jnp.float32),
    )(hv, w1c)


def _sc_gather(table, idx2d, part):
    NBUF = 3
    chunk0 = part * (NCHUNK // PARTS)

    @functools.partial(
        pl.kernel,
        mesh=plsc.VectorSubcoreMesh(core_axis_name="c", subcore_axis_name="s"),
        out_type=jax.ShapeDtypeStruct((R // PARTS, H), jnp.float32),
        scratch_types=[
            pltpu.VMEM((CPW, CHUNK), jnp.int32),
            pltpu.VMEM((NBUF, CHUNK, H), jnp.float32),
            pltpu.SemaphoreType.DMA,
            pltpu.SemaphoreType.DMA((NBUF,)),
            pltpu.SemaphoreType.DMA((NBUF,)),
        ],
    )
    def k(table_hbm, idx_hbm, out_hbm, idx_v, rows_v, sem_i, sem_g, sem_w):
        wid = lax.axis_index("s") * 2 + lax.axis_index("c")
        base = wid * CPW
        pltpu.async_copy(idx_hbm.at[pl.ds(chunk0 + base, CPW)], idx_v,
                         sem_i).wait()

        def start_gather(t):
            return pltpu.async_copy(table_hbm.at[idx_v.at[t]],
                                    rows_v.at[t % NBUF], sem_g.at[t % NBUF])

        gh = {t: start_gather(t) for t in range(min(NBUF, CPW))}
        wh = {}
        for t in range(CPW):
            b = t % NBUF
            gh[t].wait()
            wh[t] = pltpu.async_copy(
                rows_v.at[b], out_hbm.at[pl.ds((base + t) * CHUNK, CHUNK)],
                sem_w.at[b])
            if t + NBUF < CPW:
                wh[t].wait()
                gh[t + NBUF] = start_gather(t + NBUF)
        for t in range(max(0, CPW - NBUF), CPW):
            wh[t].wait()

    return k(table, idx2d)


def _main_body(hv_ref, he_ref, g1_ref, ma_ref, mv_ref,
               w1a_ref, b1_ref, w1b_ref, w2_ref, b2_ref, w3_ref, b3_ref,
               wdin_ref, bdin_ref, wdout_ref, bdout_ref,
               ln1g_ref, ln1b_ref, ln2g_ref, ln2b_ref,
               w11a_ref, b11_ref, w11c_ref,
               hv2_ref, q2_ref, pre2_ref):
    hv = hv_ref[...]
    pre1 = _dot16(hv, w1a_ref[...]) + b1_ref[...]
    he16 = he_ref[...].reshape(EDGE_BLK, H)
    x = _dot16(he16, w1b_ref[...]) + g1_ref[...]
    x = (x.reshape(NODE_BLK, K, H) + pre1[:, None, :]).reshape(EDGE_BLK, H)
    x = _gelu16(x)
    x = _gelu16(_dot16(x, w2_ref[...]) + b2_ref[...])
    m = _dot16(x, w3_ref[...]) + b3_ref[...]
    m3 = m.reshape(NODE_BLK, K, H) * ma_ref[...].reshape(NODE_BLK, K)[:, :, None]
    dh = jnp.sum(m3, axis=1)
    hv2 = _ln(hv + dh, ln1g_ref[...], ln1b_ref[...])
    ffn = _dot16(_gelu16(_dot16(hv2, wdin_ref[...]) + bdin_ref[...]),
                 wdout_ref[...]) + bdout_ref[...]
    hv2 = _ln(hv2 + ffn, ln2g_ref[...], ln2b_ref[...])
    hv2 = hv2 * mv_ref[...]
    hv2_ref[...] = hv2
    q2_ref[...] = jnp.dot(hv2, w11c_ref[...])
    pre2_ref[...] = _dot16(hv2, w11a_ref[...]) + b11_ref[...]


def _tc_main(hv, he4, g1h, ma, mv, weights, part):
    off = part * PBLK
    node_h = pl.BlockSpec((NODE_BLK, H), lambda i, off=off: (i + off, 0))
    he_spec = pl.BlockSpec((1, NODE_BLK, K, H),
                           lambda i, off=off: (0, i + off, 0, 0))
    gh_spec = pl.BlockSpec((EDGE_BLK, H), lambda i: (i, 0))
    out_node = pl.BlockSpec((NODE_BLK, H), lambda i: (i, 0))

    def full(a):
        return pl.BlockSpec(a.shape, lambda i: (0,) * a.ndim)

    in_specs = [
        node_h, he_spec, gh_spec,
        pl.BlockSpec((1, NODE_BLK, K), lambda i, off=off: (0, i + off, 0)),
        pl.BlockSpec((NODE_BLK, 1), lambda i, off=off: (i + off, 0)),
    ] + [full(w) for w in weights]
    out_specs = [out_node, out_node, out_node]
    out_shape = [jax.ShapeDtypeStruct((N // PARTS, H), jnp.float32)] * 3
    return pl.pallas_call(
        _main_body,
        grid=(PBLK,),
        in_specs=in_specs,
        out_specs=out_specs,
        out_shape=out_shape,
        compiler_params=pltpu.CompilerParams(
            dimension_semantics=("arbitrary",)),
    )(hv, he4, g1h, ma, mv, *weights)


def _edge_body(he_ref, g2_ref, pre2_ref, w11b_ref, w12_ref, b12_ref,
               w13_ref, b13_ref, ln3g_ref, ln3b_ref, he2_ref):
    he16 = he_ref[...].reshape(EDGE_BLK, H)
    x = _dot16(he16, w11b_ref[...]) + g2_ref[...]
    x = (x.reshape(NODE_BLK, K, H) + pre2_ref[...][:, None, :]).reshape(EDGE_BLK, H)
    x = _gelu16(x)
    x = _gelu16(_dot16(x, w12_ref[...]) + b12_ref[...])
    m = _dot16(x, w13_ref[...]) + b13_ref[...]
    he2 = _ln(he16.astype(jnp.float32) + m, ln3g_ref[...], ln3b_ref[...])
    he2_ref[...] = he2.reshape(1, NODE_BLK, K, H)


def _edge_body_aliased(_alias_ref, *rest):
    _edge_body(*rest)


def _tc_edge(he4, g2p, pre2p, weights, part, he2_prev):
    off = part * PBLK
    he_spec = pl.BlockSpec((1, NODE_BLK, K, H),
                           lambda i, off=off: (0, i + off, 0, 0))
    gh_spec = pl.BlockSpec((EDGE_BLK, H), lambda i: (i, 0))
    node_h = pl.BlockSpec((NODE_BLK, H), lambda i: (i, 0))
    out_spec = pl.BlockSpec((1, NODE_BLK, K, H),
                            lambda i, off=off: (0, i + off, 0, 0))

    def full(a):
        return pl.BlockSpec(a.shape, lambda i: (0,) * a.ndim)

    in_specs = [he_spec, gh_spec, node_h] + [full(w) for w in weights]
    body = _edge_body
    args = (he4, g2p, pre2p) + tuple(weights)
    aliases = {}
    if he2_prev is not None:
        in_specs = [pl.BlockSpec(memory_space=pltpu.MemorySpace.HBM)] + in_specs
        body = _edge_body_aliased
        args = (he2_prev,) + args
        aliases = {0: 0}
    return pl.pallas_call(
        body,
        grid=(PBLK,),
        in_specs=in_specs,
        out_specs=out_spec,
        out_shape=jax.ShapeDtypeStruct((1, N, K, H), jnp.float32),
        input_output_aliases=aliases,
        compiler_params=pltpu.CompilerParams(
            dimension_semantics=("arbitrary",)),
    )(*args)


def kernel(h_V, h_E, mask_V, mask_attend, W1, b1, W2, b2, W3, b3,
           Wd_in, bd_in, Wd_out, bd_out, W11, b11, W12, b12, W13, b13,
           ln1_g, ln1_b, ln2_g, ln2_b, ln3_g, ln3_b, E_idx):
    hv = h_V.reshape(N, H)
    ma = mask_attend
    mv = mask_V.reshape(N, 1)
    idx2d = E_idx.reshape(NCHUNK, CHUNK).astype(jnp.int32)

    w1a, w1b, w1c = W1[:H], W1[H:2 * H], W1[2 * H:]
    w11a, w11b, w11c = W11[:H], W11[H:2 * H], W11[2 * H:]
    row = lambda v: v.reshape(1, -1)

    main_w = (w1a, row(b1), w1b, W2, row(b2), W3 * (1.0 / SCALE),
              row(b3) * (1.0 / SCALE),
              Wd_in, row(bd_in), Wd_out, row(bd_out),
              row(ln1_g), row(ln1_b), row(ln2_g), row(ln2_b),
              w11a, row(b11), w11c)
    edge_w = (w11b, W12, row(b12), W13, row(b13), row(ln3_g), row(ln3_b))

    q1 = _tc_prep(hv, w1c)
    g1 = [_sc_gather(q1, idx2d, p) for p in range(PARTS)]
    mains = [_tc_main(hv, h_E, g1[p], ma, mv, main_w, p)
             for p in range(PARTS)]
    q2 = jnp.concatenate([m[1] for m in mains], axis=0)
    g2 = [_sc_gather(q2, idx2d, p) for p in range(PARTS)]
    he2 = None
    for p in range(PARTS):
        he2 = _tc_edge(h_E, g2[p], mains[p][2], edge_w, p, he2)
    hv2 = jnp.concatenate([m[0] for m in mains], axis=0)
    return (hv2.reshape(1, N, H), he2)


# speedup vs baseline: 1.2159x; 1.0081x over previous
import functools

import jax
import jax.numpy as jnp
from jax import lax
from jax.experimental import pallas as pl
from jax.experimental.pallas import tpu as pltpu
from jax.experimental.pallas import tpu_sc as plsc

N = 2048
K = 32
H = 128
R = N * K
FF = 4 * H
SCALE = 30.0

NODE_BLK = 256
EDGE_BLK = NODE_BLK * K
NBLK = N // NODE_BLK
PARTS = 4
PBLK = NBLK // PARTS

CHUNK = 128
NWORKERS = 32
NCHUNK = R // CHUNK
CPW = NCHUNK // PARTS // NWORKERS

_SQRT_HALF = 0.7071067811865476


def _gelu(x):
    return 0.5 * x * (1.0 + lax.erf(x * _SQRT_HALF))


def _gelu16(x):
    x = x.astype(jnp.bfloat16)
    return (jnp.bfloat16(0.5) * x
            * (jnp.bfloat16(1.0)
               + lax.erf(x * jnp.bfloat16(_SQRT_HALF))))


def _dot16(a, b):
    return jnp.dot(a.astype(jnp.bfloat16), b.astype(jnp.bfloat16),
                   preferred_element_type=jnp.float32)


def _ln(x, g, b):
    mu = jnp.mean(x, axis=-1, keepdims=True)
    s2 = jnp.mean(x * x, axis=-1, keepdims=True)
    r = lax.rsqrt(s2 - mu * mu + 1e-5)
    return (x - mu) * r * g + b


def _prep_body(hv_ref, w_ref, q_ref):
    q_ref[...] = jnp.dot(hv_ref[...], w_ref[...])


def _tc_prep(hv, w1c):
    return pl.pallas_call(
        _prep_body,
        out_shape=jax.ShapeDtypeStruct((N, H), jnp.float32),
    )(hv, w1c)


def _sc_gather(table, idx2d, part):
    NBUF = 3
    chunk0 = part * (NCHUNK // PARTS)

    @functools.partial(
        pl.kernel,
        mesh=plsc.VectorSubcoreMesh(core_axis_name="c", subcore_axis_name="s"),
        out_type=jax.ShapeDtypeStruct((R // PARTS, H), jnp.float32),
        scratch_types=[
            pltpu.VMEM((CPW, CHUNK), jnp.int32),
            pltpu.VMEM((NBUF, CHUNK, H), jnp.float32),
            pltpu.SemaphoreType.DMA,
            pltpu.SemaphoreType.DMA((NBUF,)),
            pltpu.SemaphoreType.DMA((NBUF,)),
        ],
    )
    def k(table_hbm, idx_hbm, out_hbm, idx_v, rows_v, sem_i, sem_g, sem_w):
        wid = lax.axis_index("s") * 2 + lax.axis_index("c")
        base = wid * CPW
        pltpu.async_copy(idx_hbm.at[pl.ds(chunk0 + base, CPW)], idx_v,
                         sem_i).wait()

        def start_gather(t):
            return pltpu.async_copy(table_hbm.at[idx_v.at[t]],
                                    rows_v.at[t % NBUF], sem_g.at[t % NBUF])

        gh = {t: start_gather(t) for t in range(min(NBUF, CPW))}
        wh = {}
        for t in range(CPW):
            b = t % NBUF
            gh[t].wait()
            wh[t] = pltpu.async_copy(
                rows_v.at[b], out_hbm.at[pl.ds((base + t) * CHUNK, CHUNK)],
                sem_w.at[b])
            if t + NBUF < CPW:
                wh[t].wait()
                gh[t + NBUF] = start_gather(t + NBUF)
        for t in range(max(0, CPW - NBUF), CPW):
            wh[t].wait()

    return k(table, idx2d)


def _main_body(hv_ref, he_ref, g1_ref, ma_ref, mv_ref,
               w1a_ref, b1_ref, w1b_ref, w2_ref, b2_ref, w3_ref, b3_ref,
               wdin_ref, bdin_ref, wdout_ref, bdout_ref,
               ln1g_ref, ln1b_ref, ln2g_ref, ln2b_ref,
               w11a_ref, b11_ref, w11c_ref,
               hv2_ref, q2_ref, pre2_ref):
    hv = hv_ref[...]
    pre1 = _dot16(hv, w1a_ref[...]) + b1_ref[...]
    he16 = he_ref[...].reshape(EDGE_BLK, H)
    x = _dot16(he16, w1b_ref[...]) + g1_ref[...]
    x = (x.reshape(NODE_BLK, K, H) + pre1[:, None, :]).reshape(EDGE_BLK, H)
    x = _gelu16(x)
    x = _gelu16(_dot16(x, w2_ref[...]) + b2_ref[...])
    m = _dot16(x, w3_ref[...]) + b3_ref[...]
    m3 = m.reshape(NODE_BLK, K, H) * ma_ref[...].reshape(NODE_BLK, K)[:, :, None]
    dh = jnp.sum(m3, axis=1)
    hv2 = _ln(hv + dh, ln1g_ref[...], ln1b_ref[...])
    ffn = _dot16(_gelu16(_dot16(hv2, wdin_ref[...]) + bdin_ref[...]),
                 wdout_ref[...]) + bdout_ref[...]
    hv2 = _ln(hv2 + ffn, ln2g_ref[...], ln2b_ref[...])
    hv2 = hv2 * mv_ref[...]
    hv2_ref[...] = hv2
    q2_ref[...] = jnp.dot(hv2, w11c_ref[...])
    pre2_ref[...] = _dot16(hv2, w11a_ref[...]) + b11_ref[...]


def _main_body_aliased(_alias_ref, *rest):
    _main_body(*rest)


def _tc_main(hv, he4, g1h, ma, mv, weights, part, q2_prev):
    off = part * PBLK
    node_h = pl.BlockSpec((NODE_BLK, H), lambda i, off=off: (i + off, 0))
    he_spec = pl.BlockSpec((1, NODE_BLK, K, H),
                           lambda i, off=off: (0, i + off, 0, 0))
    gh_spec = pl.BlockSpec((EDGE_BLK, H), lambda i: (i, 0))
    out_node = pl.BlockSpec((NODE_BLK, H), lambda i: (i, 0))
    q2_spec = pl.BlockSpec((NODE_BLK, H), lambda i, off=off: (i + off, 0))

    def full(a):
        return pl.BlockSpec(a.shape, lambda i: (0,) * a.ndim)

    in_specs = [
        node_h, he_spec, gh_spec,
        pl.BlockSpec((1, NODE_BLK, K), lambda i, off=off: (0, i + off, 0)),
        pl.BlockSpec((NODE_BLK, 1), lambda i, off=off: (i + off, 0)),
    ] + [full(w) for w in weights]
    out_specs = [out_node, q2_spec, out_node]
    out_shape = [jax.ShapeDtypeStruct((N // PARTS, H), jnp.float32),
                 jax.ShapeDtypeStruct((N, H), jnp.float32),
                 jax.ShapeDtypeStruct((N // PARTS, H), jnp.float32)]
    body = _main_body
    args = (hv, he4, g1h, ma, mv) + tuple(weights)
    aliases = {}
    if q2_prev is not None:
        in_specs = [pl.BlockSpec(memory_space=pltpu.MemorySpace.HBM)] + in_specs
        body = _main_body_aliased
        args = (q2_prev,) + args
        aliases = {0: 1}
    return pl.pallas_call(
        body,
        grid=(PBLK,),
        in_specs=in_specs,
        out_specs=out_specs,
        out_shape=out_shape,
        input_output_aliases=aliases,
        compiler_params=pltpu.CompilerParams(
            dimension_semantics=("arbitrary",)),
    )(*args)


def _edge_body(he_ref, g2_ref, pre2_ref, w11b_ref, w12_ref, b12_ref,
               w13_ref, b13_ref, ln3g_ref, ln3b_ref, he2_ref):
    he16 = he_ref[...].reshape(EDGE_BLK, H)
    x = _dot16(he16, w11b_ref[...]) + g2_ref[...]
    x = (x.reshape(NODE_BLK, K, H) + pre2_ref[...][:, None, :]).reshape(EDGE_BLK, H)
    x = _gelu16(x)
    x = _gelu16(_dot16(x, w12_ref[...]) + b12_ref[...])
    m = _dot16(x, w13_ref[...]) + b13_ref[...]
    he2 = _ln(he16.astype(jnp.float32) + m, ln3g_ref[...], ln3b_ref[...])
    he2_ref[...] = he2.reshape(1, NODE_BLK, K, H)


def _edge_body_aliased(_alias_ref, *rest):
    _edge_body(*rest)


def _tc_edge(he4, g2p, pre2p, weights, part, he2_prev):
    off = part * PBLK
    he_spec = pl.BlockSpec((1, NODE_BLK, K, H),
                           lambda i, off=off: (0, i + off, 0, 0))
    gh_spec = pl.BlockSpec((EDGE_BLK, H), lambda i: (i, 0))
    node_h = pl.BlockSpec((NODE_BLK, H), lambda i: (i, 0))
    out_spec = pl.BlockSpec((1, NODE_BLK, K, H),
                            lambda i, off=off: (0, i + off, 0, 0))

    def full(a):
        return pl.BlockSpec(a.shape, lambda i: (0,) * a.ndim)

    in_specs = [he_spec, gh_spec, node_h] + [full(w) for w in weights]
    body = _edge_body
    args = (he4, g2p, pre2p) + tuple(weights)
    aliases = {}
    if he2_prev is not None:
        in_specs = [pl.BlockSpec(memory_space=pltpu.MemorySpace.HBM)] + in_specs
        body = _edge_body_aliased
        args = (he2_prev,) + args
        aliases = {0: 0}
    return pl.pallas_call(
        body,
        grid=(PBLK,),
        in_specs=in_specs,
        out_specs=out_spec,
        out_shape=jax.ShapeDtypeStruct((1, N, K, H), jnp.float32),
        input_output_aliases=aliases,
        compiler_params=pltpu.CompilerParams(
            dimension_semantics=("arbitrary",)),
    )(*args)


def kernel(h_V, h_E, mask_V, mask_attend, W1, b1, W2, b2, W3, b3,
           Wd_in, bd_in, Wd_out, bd_out, W11, b11, W12, b12, W13, b13,
           ln1_g, ln1_b, ln2_g, ln2_b, ln3_g, ln3_b, E_idx):
    hv = h_V.reshape(N, H)
    ma = mask_attend
    mv = mask_V.reshape(N, 1)
    idx2d = E_idx.reshape(NCHUNK, CHUNK).astype(jnp.int32)

    w1a, w1b, w1c = W1[:H], W1[H:2 * H], W1[2 * H:]
    w11a, w11b, w11c = W11[:H], W11[H:2 * H], W11[2 * H:]
    row = lambda v: v.reshape(1, -1)

    main_w = (w1a, row(b1), w1b, W2, row(b2), W3 * (1.0 / SCALE),
              row(b3) * (1.0 / SCALE),
              Wd_in, row(bd_in), Wd_out, row(bd_out),
              row(ln1_g), row(ln1_b), row(ln2_g), row(ln2_b),
              w11a, row(b11), w11c)
    edge_w = (w11b, W12, row(b12), W13, row(b13), row(ln3_g), row(ln3_b))

    q1 = _tc_prep(hv, w1c)
    g1 = [_sc_gather(q1, idx2d, p) for p in range(PARTS)]
    mains = []
    q2 = None
    for p in range(PARTS):
        mains.append(_tc_main(hv, h_E, g1[p], ma, mv, main_w, p, q2))
        q2 = mains[p][1]
    g2 = [_sc_gather(q2, idx2d, p) for p in range(PARTS)]
    he2 = None
    for p in range(PARTS):
        he2 = _tc_edge(h_E, g2[p], mains[p][2], edge_w, p, he2)
    hv2 = jnp.concatenate([m[0] for m in mains], axis=0)
    return (hv2.reshape(1, N, H), he2)
